# baseline (speedup 1.0000x reference)
import functools

import jax
import jax.numpy as jnp
from jax.experimental import pallas as pl
from jax.experimental.pallas import tpu as pltpu

K_NB = 16
BQ = 256


def _dot(a, b):
    return jax.lax.dot_general(a, b, (((1,), (0,)), ((), ())),
                               preferred_element_type=jnp.float32)


def _tc_body(y3_ref, qc_ref, f3_ref, kW0_ref, kb0_ref, kW1_ref, kb1_ref,
             pW0_ref, pb0_ref, pW1_ref, pb1_ref, out_ref):
    hid = kW0_ref.shape[1]
    in_ch = kW1_ref.shape[1]
    y = y3_ref[...].reshape(K_NB * BQ, 2)
    ypre = _dot(y, kW0_ref[0:2, :])
    qpre = _dot(qc_ref[...], kW0_ref[2:4, :]) + kb0_ref[...]
    h = jax.nn.gelu(ypre.reshape(K_NB, BQ, hid) + qpre[None, :, :])
    kern = (_dot(h.reshape(K_NB * BQ, hid), kW1_ref[...]) + kb1_ref[...])
    kern = kern.reshape(K_NB, BQ, in_ch)
    dec = jnp.sum(kern * f3_ref[...], axis=0)
    h2 = jax.nn.gelu(_dot(dec, pW0_ref[...]) + pb0_ref[...])
    out_ref[...] = _dot(h2, pW1_ref[...]) + pb1_ref[...]


def kernel(latent_tokens_coord, rndata, query_coord, nbr_index, nbr_row_splits,
           kW0, kb0, kW1, kb1, pW0, pb0, pW1, pb1):
    Q = query_coord.shape[0]
    K = nbr_index.shape[0] // Q
    assert K == K_NB
    out_ch = pW1.shape[1]
    nb = (Q + BQ - 1) // BQ
    q_pad = nb * BQ

    idx = nbr_index.reshape(Q, K).T
    idx = jnp.pad(idx, ((0, 0), (0, q_pad - Q)))
    y3 = jnp.take(latent_tokens_coord, idx, axis=0)
    f3 = jnp.take(rndata[0], idx, axis=0)
    qcp = jnp.pad(query_coord, ((0, q_pad - Q), (0, 0)))

    full = lambda b: (0, 0)
    out = pl.pallas_call(
        _tc_body,
        grid=(nb,),
        in_specs=[
            pl.BlockSpec((K, BQ, 2), lambda b: (0, b, 0)),
            pl.BlockSpec((BQ, 2), lambda b: (b, 0)),
            pl.BlockSpec((K, BQ, rndata.shape[2]), lambda b: (0, b, 0)),
            pl.BlockSpec(kW0.shape, full),
            pl.BlockSpec((1, kb0.shape[0]), full),
            pl.BlockSpec(kW1.shape, full),
            pl.BlockSpec((1, kb1.shape[0]), full),
            pl.BlockSpec(pW0.shape, full),
            pl.BlockSpec((1, pb0.shape[0]), full),
            pl.BlockSpec(pW1.shape, full),
            pl.BlockSpec((1, pb1.shape[0]), full),
        ],
        out_specs=pl.BlockSpec((BQ, out_ch), lambda b: (b, 0)),
        out_shape=jax.ShapeDtypeStruct((q_pad, out_ch), jnp.float32),
    )(y3, qcp, f3, kW0, kb0.reshape(1, -1), kW1, kb1.reshape(1, -1),
      pW0, pb0.reshape(1, -1), pW1, pb1.reshape(1, -1))
    return out[None, :Q, :]


# speedup vs baseline: 5.8569x; 5.8569x over previous
import functools

import jax
import jax.numpy as jnp
from jax import lax
from jax.experimental import pallas as pl
from jax.experimental.pallas import tpu as pltpu
from jax.experimental.pallas import tpu_sc as plsc

K_NB = 16
BQ = 256
NW = 32
CQ = 112
D_ROW = 34


def _sc_body(idx_hbm, table_hbm, fy_hbm, idx_v, klist, fbuf, sem):
    n_chunks = idx_hbm.shape[0] // (NW * CQ * K_NB)
    cid = lax.axis_index("c")
    sid = lax.axis_index("s")
    wid = sid * 2 + cid
    qbase0 = wid * (n_chunks * CQ)
    lanes = lax.broadcasted_iota(jnp.int32, (16,), 0)

    def chunk_body(ci, carry):
        qbase = qbase0 + ci * CQ
        pltpu.sync_copy(idx_hbm.at[pl.ds(qbase * K_NB, CQ * K_NB)], idx_v)

        def regroup(g, c2):
            base = g * (16 * K_NB)
            for k in range(K_NB):
                vec = plsc.load_gather(idx_v, [lanes * K_NB + (base + k)])
                klist[k, pl.ds(g * 16, 16)] = vec
            return c2

        lax.fori_loop(0, CQ // 16, regroup, 0, unroll=True)
        for k in range(K_NB):
            pltpu.async_copy(table_hbm.at[klist.at[k]], fbuf.at[k], sem).wait()
        pltpu.sync_copy(fbuf, fy_hbm.at[:, pl.ds(qbase, CQ), :])
        return carry

    lax.fori_loop(0, n_chunks, chunk_body, 0)


def _sc_gather(idx_pad, table, q_pad):
    mesh = plsc.VectorSubcoreMesh(core_axis_name="c", subcore_axis_name="s")
    return pl.kernel(
        _sc_body,
        out_type=jax.ShapeDtypeStruct((K_NB, q_pad, D_ROW), jnp.float32),
        mesh=mesh,
        scratch_types=[
            pltpu.VMEM((CQ * K_NB,), jnp.int32),
            pltpu.VMEM((K_NB, CQ), jnp.int32),
            pltpu.VMEM((K_NB, CQ, D_ROW), jnp.float32),
            pltpu.SemaphoreType.DMA,
        ],
        compiler_params=pltpu.CompilerParams(needs_layout_passes=False,
                                             use_tc_tiling_on_sc=False),
    )(idx_pad, table)


def _dot(a, b):
    return jax.lax.dot_general(a, b, (((1,), (0,)), ((), ())),
                               preferred_element_type=jnp.float32)


def _tc_body(fy_ref, qc_ref, W0e_ref, kW0q_ref, kb0_ref, kW1_ref, kb1_ref,
             pW0_ref, pb0_ref, pW1_ref, pb1_ref, out_ref):
    hid = W0e_ref.shape[1]
    in_ch = kW1_ref.shape[1]
    fy = fy_ref[...]
    ypre = _dot(fy.reshape(K_NB * BQ, D_ROW), W0e_ref[...])
    qpre = _dot(qc_ref[...], kW0q_ref[...]) + kb0_ref[...]
    h = jax.nn.gelu(ypre.reshape(K_NB, BQ, hid) + qpre[None, :, :])
    kern = (_dot(h.reshape(K_NB * BQ, hid), kW1_ref[...]) + kb1_ref[...])
    kern = kern.reshape(K_NB, BQ, in_ch)
    dec = jnp.sum(kern * fy[:, :, :in_ch], axis=0)
    h2 = jax.nn.gelu(_dot(dec, pW0_ref[...]) + pb0_ref[...])
    out_ref[...] = _dot(h2, pW1_ref[...]) + pb1_ref[...]


def kernel(latent_tokens_coord, rndata, query_coord, nbr_index, nbr_row_splits,
           kW0, kb0, kW1, kb1, pW0, pb0, pW1, pb1):
    Q = query_coord.shape[0]
    K = nbr_index.shape[0] // Q
    assert K == K_NB
    n_lat = latent_tokens_coord.shape[0]
    in_ch = rndata.shape[2]
    out_ch = pW1.shape[1]
    nb = (Q + BQ - 1) // BQ
    q_pad = nb * BQ
    assert q_pad % (NW * CQ) == 0

    idx_pad = jnp.pad(nbr_index, (0, (q_pad - Q) * K))
    table = jnp.concatenate([rndata[0], latent_tokens_coord], axis=1)
    fy3 = _sc_gather(idx_pad, table, q_pad)

    W0e = jnp.concatenate([jnp.zeros((in_ch, kW0.shape[1]), jnp.float32),
                           kW0[0:2, :]], axis=0)
    qcp = jnp.pad(query_coord, ((0, q_pad - Q), (0, 0)))

    full = lambda b: (0, 0)
    out = pl.pallas_call(
        _tc_body,
        grid=(nb,),
        in_specs=[
            pl.BlockSpec((K, BQ, D_ROW), lambda b: (0, b, 0)),
            pl.BlockSpec((BQ, 2), lambda b: (b, 0)),
            pl.BlockSpec(W0e.shape, full),
            pl.BlockSpec((2, kW0.shape[1]), full),
            pl.BlockSpec((1, kb0.shape[0]), full),
            pl.BlockSpec(kW1.shape, full),
            pl.BlockSpec((1, kb1.shape[0]), full),
            pl.BlockSpec(pW0.shape, full),
            pl.BlockSpec((1, pb0.shape[0]), full),
            pl.BlockSpec(pW1.shape, full),
            pl.BlockSpec((1, pb1.shape[0]), full),
        ],
        out_specs=pl.BlockSpec((BQ, out_ch), lambda b: (b, 0)),
        out_shape=jax.ShapeDtypeStruct((q_pad, out_ch), jnp.float32),
    )(fy3, qcp, W0e, kW0[2:4, :], kb0.reshape(1, -1), kW1, kb1.reshape(1, -1),
      pW0, pb0.reshape(1, -1), pW1, pb1.reshape(1, -1))
    return out[None, :Q, :]


# speedup vs baseline: 7.6849x; 1.3121x over previous
import functools

import jax
import jax.numpy as jnp
from jax import lax
from jax.experimental import pallas as pl
from jax.experimental.pallas import tpu as pltpu
from jax.experimental.pallas import tpu_sc as plsc

K_NB = 16
PQ = 3
BQ = 384
NW = 32
CQ = 48
IN_CH = 32
HID = 64
Q_PAD = 50688
F_ROWS = Q_PAD // PQ


def _sc_body(idx_hbm, table_hbm, ltc_hbm, qc_hbm, rows_hbm,
             idx_v, klist, ltc_v, qc_v, gbuf, rbuf, sem_g):
    n_chunks = idx_hbm.shape[0] // (NW * CQ * K_NB)
    cid = lax.axis_index("c")
    sid = lax.axis_index("s")
    wid = sid * 2 + cid
    qbase0 = wid * (n_chunks * CQ)
    lanes = lax.broadcasted_iota(jnp.int32, (16,), 0)
    zeros16 = jnp.zeros((16,), jnp.float32)

    pltpu.sync_copy(ltc_hbm, ltc_v)
    for k in range(K_NB):
        krow = jnp.full((16,), k, jnp.int32)
        for r in range(CQ // PQ):
            rrow = jnp.full((16,), r, jnp.int32)
            plsc.store_scatter(rbuf, [krow, rrow, 108 + lanes], zeros16)
            plsc.store_scatter(rbuf, [krow, rrow, 112 + lanes], zeros16)

    def chunk_body(ci, carry):
        qbase = qbase0 + ci * CQ
        pltpu.sync_copy(idx_hbm.at[pl.ds(qbase * K_NB, CQ * K_NB)], idx_v)
        pltpu.sync_copy(qc_hbm.at[pl.ds(qbase * 2, CQ * 2)], qc_v)

        def regroup(g, c2):
            qloc = g * 16 + lanes
            rrow = (qloc * 21846) >> 16
            lbase = 2 * (qloc - 3 * rrow)
            qcx = plsc.load_gather(qc_v, [qloc * 2])
            qcy = plsc.load_gather(qc_v, [qloc * 2 + 1])
            for k in range(K_NB):
                krow = jnp.full((16,), k, jnp.int32)
                vec = plsc.load_gather(idx_v, [qloc * K_NB + k])
                klist[k, pl.ds(g * 16, 16)] = vec
                yx = plsc.load_gather(ltc_v, [vec * 2])
                yy = plsc.load_gather(ltc_v, [vec * 2 + 1])
                plsc.store_scatter(rbuf, [krow, rrow, 96 + lbase], yx)
                plsc.store_scatter(rbuf, [krow, rrow, 97 + lbase], yy)
                plsc.store_scatter(rbuf, [krow, rrow, 102 + lbase], qcx)
                plsc.store_scatter(rbuf, [krow, rrow, 103 + lbase], qcy)
            return c2

        lax.fori_loop(0, CQ // 16, regroup, 0, unroll=True)

        descs = [
            pltpu.async_copy(table_hbm.at[klist.at[k]], gbuf.at[k], sem_g)
            for k in range(K_NB)
        ]
        for d in descs:
            d.wait()

        def repack(r, c2):
            for k in range(K_NB):
                for m in range(6):
                    rbuf[k, r, pl.ds(m * 16, 16)] = (
                        gbuf[k, PQ * r + m // 2, pl.ds((m % 2) * 16, 16)])
            return c2

        lax.fori_loop(0, CQ // PQ, repack, 0)
        pltpu.sync_copy(rbuf,
                        rows_hbm.at[:, pl.ds(qbase // PQ, CQ // PQ), :])
        return carry

    lax.fori_loop(0, n_chunks, chunk_body, 0)


def _sc_gather(idx_pad, table, ltc_flat, qc_flat):
    mesh = plsc.VectorSubcoreMesh(core_axis_name="c", subcore_axis_name="s")
    return pl.kernel(
        _sc_body,
        out_type=jax.ShapeDtypeStruct((K_NB, F_ROWS, 128), jnp.float32),
        mesh=mesh,
        scratch_types=[
            pltpu.VMEM((CQ * K_NB,), jnp.int32),
            pltpu.VMEM((K_NB, CQ), jnp.int32),
            pltpu.VMEM((ltc_flat.shape[0],), jnp.float32),
            pltpu.VMEM((CQ * 2,), jnp.float32),
            pltpu.VMEM((K_NB, CQ, IN_CH), jnp.float32),
            pltpu.VMEM((K_NB, CQ // PQ, 128), jnp.float32),
            pltpu.SemaphoreType.DMA,
        ],
        compiler_params=pltpu.CompilerParams(needs_layout_passes=False,
                                             use_tc_tiling_on_sc=False),
    )(idx_pad, table, ltc_flat, qc_flat)


def _dot(a, b):
    return jax.lax.dot_general(a, b, (((1,), (0,)), ((), ())),
                               preferred_element_type=jnp.float32)


def _tc_body(fr_ref, W0_ref, b0_ref, W1_ref, b1_ref, pW0_ref, pb0_ref,
             pW1_ref, pb1_ref, out_ref):
    nr = BQ // PQ
    fr = fr_ref[...]
    fr2 = fr.reshape(K_NB * nr, 128)
    h = jax.nn.gelu(_dot(fr2, W0_ref[...]) + b0_ref[...])
    kern = (_dot(h, W1_ref[...]) + b1_ref[...]).reshape(K_NB, nr, 128)
    dec = jnp.sum(kern * fr, axis=0)
    h2 = jax.nn.gelu(_dot(dec, pW0_ref[...]) + pb0_ref[...])
    out_ref[...] = _dot(h2, pW1_ref[...]) + pb1_ref[...]


def kernel(latent_tokens_coord, rndata, query_coord, nbr_index, nbr_row_splits,
           kW0, kb0, kW1, kb1, pW0, pb0, pW1, pb1):
    Q = query_coord.shape[0]
    K = nbr_index.shape[0] // Q
    assert K == K_NB
    out_ch = pW1.shape[1]
    proj_ch = pW0.shape[1]
    nb = Q_PAD // BQ

    idx_pad = jnp.pad(nbr_index, (0, (Q_PAD - Q) * K))
    qc_flat = jnp.pad(query_coord, ((0, Q_PAD - Q), (0, 0))).reshape(-1)
    rows = _sc_gather(idx_pad, rndata[0], latent_tokens_coord.reshape(-1),
                      qc_flat)

    W0 = jnp.zeros((128, PQ * HID), jnp.float32)
    W1 = jnp.zeros((PQ * HID, 128), jnp.float32)
    pW0b = jnp.zeros((128, PQ * proj_ch), jnp.float32)
    pW1b = jnp.zeros((PQ * proj_ch, PQ * out_ch), jnp.float32)
    for p in range(PQ):
        W0 = W0.at[96 + 2 * p:98 + 2 * p, HID * p:HID * (p + 1)].set(kW0[0:2])
        W0 = W0.at[102 + 2 * p:104 + 2 * p,
                   HID * p:HID * (p + 1)].set(kW0[2:4])
        W1 = W1.at[HID * p:HID * (p + 1),
                   IN_CH * p:IN_CH * (p + 1)].set(kW1)
        pW0b = pW0b.at[IN_CH * p:IN_CH * (p + 1),
                       proj_ch * p:proj_ch * (p + 1)].set(pW0)
        pW1b = pW1b.at[proj_ch * p:proj_ch * (p + 1),
                       out_ch * p:out_ch * (p + 1)].set(pW1)
    b0 = jnp.tile(kb0, PQ).reshape(1, -1)
    b1 = jnp.concatenate([jnp.tile(kb1, PQ),
                          jnp.zeros(128 - PQ * IN_CH)]).reshape(1, -1)
    pb0 = jnp.tile(pb0, PQ).reshape(1, -1)
    pb1 = jnp.tile(pb1, PQ).reshape(1, -1)

    full = lambda b: (0, 0)
    out = pl.pallas_call(
        _tc_body,
        grid=(nb,),
        in_specs=[
            pl.BlockSpec((K, BQ // PQ, 128), lambda b: (0, b, 0)),
            pl.BlockSpec(W0.shape, full),
            pl.BlockSpec(b0.shape, full),
            pl.BlockSpec(W1.shape, full),
            pl.BlockSpec(b1.shape, full),
            pl.BlockSpec(pW0b.shape, full),
            pl.BlockSpec(pb0.shape, full),
            pl.BlockSpec(pW1b.shape, full),
            pl.BlockSpec(pb1.shape, full),
        ],
        out_specs=pl.BlockSpec((BQ // PQ, PQ * out_ch), lambda b: (b, 0)),
        out_shape=jax.ShapeDtypeStruct((F_ROWS, PQ * out_ch), jnp.float32),
    )(rows, W0, b0, W1, b1, pW0b, pb0, pW1b, pb1)
    return out.reshape(Q_PAD, out_ch)[None, :Q, :]
